# baseline (speedup 1.0000x reference)
import jax
import jax.numpy as jnp
from jax import lax
from jax.experimental import pallas as pl
from jax.experimental.pallas import tpu as pltpu
from jax.experimental.pallas import tpu_sc as plsc

N_NODES = 10000
D = 256
N_EDGES = 160000
NC = 2
NS = 16
L = 16
NW = NC * NS
DP = 128
GSZ = 16
NBUF = 8
EPW = 5120
E_PAD = EPW * NW
G = EPW // GSZ
NOUT = G // NBUF
EPW_LAST = N_EDGES - (NW - 1) * EPW


N_SP = 10240
ROWS_PER_TILE = N_SP // NS


def _sc_body(x_hbm, si_hbm, di_hbm, out_hbm,
             si_v, di_v, x_sp, sbuf, dbuf, cs_v, out_v, sems):
    sid = lax.axis_index("s")
    wid = sid * NC + lax.axis_index("c")
    base = wid * EPW
    pltpu.sync_copy(x_hbm.at[pl.ds(sid * ROWS_PER_TILE, ROWS_PER_TILE)],
                    x_sp.at[pl.ds(sid * ROWS_PER_TILE, ROWS_PER_TILE)])
    pltpu.sync_copy(si_hbm.at[pl.ds(base, EPW)], si_v)
    pltpu.sync_copy(di_hbm.at[pl.ds(base, EPW)], di_v)
    plsc.subcore_barrier()

    def src_ref(b):
        return x_sp

    def fire(g, b):
        pltpu.async_copy(src_ref(b).at[si_v.at[pl.ds(g * GSZ, GSZ)]],
                         sbuf.at[b], sems.at[b, 0])
        pltpu.async_copy(x_sp.at[di_v.at[pl.ds(g * GSZ, GSZ)]],
                         dbuf.at[b], sems.at[b, 1])

    def wait(g, b):
        pltpu.make_async_copy(src_ref(b).at[si_v.at[pl.ds(g * GSZ, GSZ)]],
                              sbuf.at[b], sems.at[b, 0]).wait()
        pltpu.make_async_copy(x_sp.at[di_v.at[pl.ds(g * GSZ, GSZ)]],
                              dbuf.at[b], sems.at[b, 1]).wait()

    for b in range(NBUF):
        fire(b, b)

    row_sel = lax.iota(jnp.int32, L)

    def outer(i, carry):
        for b in range(NBUF):
            g = i * NBUF + b
            wait(g, b)
            for sg in range(GSZ // L):
                for ee in range(L):
                    e = sg * L + ee
                    acc0 = acc1 = None
                    for c in range(DP // L):
                        s = plsc.bitcast(sbuf[b, e, pl.ds(c * L, L)],
                                         jnp.bfloat16)
                        d = plsc.bitcast(dbuf[b, e, pl.ds(c * L, L)],
                                         jnp.bfloat16)
                        p0, p1 = plsc.unpack(
                            s * d, format=plsc.PackFormat.INTERLEAVED)
                        acc0 = p0 if acc0 is None else acc0 + p0
                        acc1 = p1 if acc1 is None else acc1 + p1
                    cs_v[pl.ds(ee * L, L)] = acc0 + acc1
                cols = [plsc.load_gather(cs_v, [row_sel * L + c])
                        for c in range(L)]
                while len(cols) > 1:
                    cols = [a + bb for a, bb in zip(cols[::2], cols[1::2])]
                out_v[pl.ds(g * GSZ + sg * L, L)] = cols[0]
            ng = g + NBUF

            @pl.when(ng < G)
            def _():
                fire(ng, b)
        return carry

    lax.fori_loop(0, NOUT, outer, 0)

    @pl.when(wid < NW - 1)
    def _():
        pltpu.sync_copy(out_v, out_hbm.at[pl.ds(base, EPW)])

    @pl.when(wid == NW - 1)
    def _():
        pltpu.sync_copy(out_v.at[pl.ds(0, EPW_LAST)],
                        out_hbm.at[pl.ds(base, EPW_LAST)])


def _run_sc(x, si, di):
    return pl.kernel(
        _sc_body,
        out_type=jax.ShapeDtypeStruct((N_EDGES,), jnp.float32),
        mesh=plsc.VectorSubcoreMesh(core_axis_name="c", subcore_axis_name="s",
                                    num_cores=NC, num_subcores=NS),
        compiler_params=pltpu.CompilerParams(needs_layout_passes=False),
        scratch_types=[
            pltpu.VMEM((EPW,), jnp.int32),
            pltpu.VMEM((EPW,), jnp.int32),
            pltpu.VMEM_SHARED((N_SP, DP), jnp.int32),
            pltpu.VMEM((NBUF, GSZ, DP), jnp.int32),
            pltpu.VMEM((NBUF, GSZ, DP), jnp.int32),
            pltpu.VMEM((L * L,), jnp.float32),
            pltpu.VMEM((EPW,), jnp.float32),
            pltpu.SemaphoreType.DMA((NBUF, 2)),
        ],
    )(x, si, di)


@jax.jit
def kernel(x, edge_label_index):
    u = lax.bitcast_convert_type(x, jnp.uint32)

    def _rne_hi16(v):
        return (v + 0x7FFF + ((v >> 16) & 1)) >> 16

    lo = _rne_hi16(u[:, :DP])
    hi = _rne_hi16(u[:, DP:])
    x_p = lax.bitcast_convert_type(lo | (hi << 16), jnp.int32)
    x_p = jnp.pad(x_p, ((0, N_SP - N_NODES), (0, 0)))
    eli = edge_label_index.astype(jnp.int32)
    pad = E_PAD - N_EDGES
    si = jnp.pad(eli[0], (0, pad))
    di = jnp.pad(eli[1], (0, pad))
    out = _run_sc(x_p, si, di)
    return out.reshape(-1, 1)


# speedup vs baseline: 1.0932x; 1.0932x over previous
import jax
import jax.numpy as jnp
from jax import lax
from jax.experimental import pallas as pl
from jax.experimental.pallas import tpu as pltpu
from jax.experimental.pallas import tpu_sc as plsc

N_NODES = 10000
D = 256
N_EDGES = 160000
NC = 2
NS = 16
L = 16
NW = NC * NS
DP = 128
GSZ = 32
NBUF = 4
EPW = 5120
E_PAD = EPW * NW
G = EPW // GSZ
NOUT = G // NBUF
EPW_LAST = N_EDGES - (NW - 1) * EPW


N_SP = 10240
ROWS_PER_TILE = N_SP // NS


def _sc_body(x_hbm, si_hbm, di_hbm, out_hbm,
             si_v, di_v, x_sp, sbuf, dbuf, cs_v, out_v, sems, stage_sem):
    sid = lax.axis_index("s")
    wid = sid * NC + lax.axis_index("c")
    base = wid * EPW
    stripe = pltpu.async_copy(
        x_hbm.at[pl.ds(sid * ROWS_PER_TILE, ROWS_PER_TILE)],
        x_sp.at[pl.ds(sid * ROWS_PER_TILE, ROWS_PER_TILE)], stage_sem)
    pltpu.sync_copy(si_hbm.at[pl.ds(base, EPW)], si_v)
    pltpu.sync_copy(di_hbm.at[pl.ds(base, EPW)], di_v)

    def src_ref(b):
        return x_sp

    def fire(g, b, src=None):
        src = src_ref(b) if src is None else src
        pltpu.async_copy(src.at[si_v.at[pl.ds(g * GSZ, GSZ)]],
                         sbuf.at[b], sems.at[b, 0])
        pltpu.async_copy(src.at[di_v.at[pl.ds(g * GSZ, GSZ)]],
                         dbuf.at[b], sems.at[b, 1])

    def wait(g, b):
        pltpu.make_async_copy(src_ref(b).at[si_v.at[pl.ds(g * GSZ, GSZ)]],
                              sbuf.at[b], sems.at[b, 0]).wait()
        pltpu.make_async_copy(x_sp.at[di_v.at[pl.ds(g * GSZ, GSZ)]],
                              dbuf.at[b], sems.at[b, 1]).wait()

    for b in range(NBUF):
        fire(b, b, src=x_hbm)
    stripe.wait()
    plsc.subcore_barrier()

    row_sel = lax.iota(jnp.int32, L)

    def outer(i, carry):
        for b in range(NBUF):
            g = i * NBUF + b
            wait(g, b)
            for sg in range(GSZ // L):
                for ee in range(L):
                    e = sg * L + ee
                    acc0 = acc1 = None
                    for c in range(DP // L):
                        s = plsc.bitcast(sbuf[b, e, pl.ds(c * L, L)],
                                         jnp.bfloat16)
                        d = plsc.bitcast(dbuf[b, e, pl.ds(c * L, L)],
                                         jnp.bfloat16)
                        p0, p1 = plsc.unpack(
                            s * d, format=plsc.PackFormat.INTERLEAVED)
                        acc0 = p0 if acc0 is None else acc0 + p0
                        acc1 = p1 if acc1 is None else acc1 + p1
                    cs_v[pl.ds(ee * L, L)] = acc0 + acc1
                cols = [plsc.load_gather(cs_v, [row_sel * L + c])
                        for c in range(L)]
                while len(cols) > 1:
                    cols = [a + bb for a, bb in zip(cols[::2], cols[1::2])]
                out_v[pl.ds(g * GSZ + sg * L, L)] = cols[0]
            ng = g + NBUF

            @pl.when(ng < G)
            def _():
                fire(ng, b)
        return carry

    lax.fori_loop(0, NOUT, outer, 0)

    @pl.when(wid < NW - 1)
    def _():
        pltpu.sync_copy(out_v, out_hbm.at[pl.ds(base, EPW)])

    @pl.when(wid == NW - 1)
    def _():
        pltpu.sync_copy(out_v.at[pl.ds(0, EPW_LAST)],
                        out_hbm.at[pl.ds(base, EPW_LAST)])


def _run_sc(x, si, di):
    return pl.kernel(
        _sc_body,
        out_type=jax.ShapeDtypeStruct((N_EDGES,), jnp.float32),
        mesh=plsc.VectorSubcoreMesh(core_axis_name="c", subcore_axis_name="s",
                                    num_cores=NC, num_subcores=NS),
        compiler_params=pltpu.CompilerParams(needs_layout_passes=False),
        scratch_types=[
            pltpu.VMEM((EPW,), jnp.int32),
            pltpu.VMEM((EPW,), jnp.int32),
            pltpu.VMEM_SHARED((N_SP, DP), jnp.int32),
            pltpu.VMEM((NBUF, GSZ, DP), jnp.int32),
            pltpu.VMEM((NBUF, GSZ, DP), jnp.int32),
            pltpu.VMEM((L * L,), jnp.float32),
            pltpu.VMEM((EPW,), jnp.float32),
            pltpu.SemaphoreType.DMA((NBUF, 2)),
            pltpu.SemaphoreType.DMA,
        ],
    )(x, si, di)


@jax.jit
def kernel(x, edge_label_index):
    u = lax.bitcast_convert_type(x, jnp.uint32)

    def _rne_hi16(v):
        return (v + 0x7FFF + ((v >> 16) & 1)) >> 16

    lo = _rne_hi16(u[:, :DP])
    hi = _rne_hi16(u[:, DP:])
    x_p = lax.bitcast_convert_type(lo | (hi << 16), jnp.int32)
    x_p = jnp.pad(x_p, ((0, N_SP - N_NODES), (0, 0)))
    eli = edge_label_index.astype(jnp.int32)
    pad = E_PAD - N_EDGES
    si = jnp.pad(eli[0], (0, pad))
    di = jnp.pad(eli[1], (0, pad))
    out = _run_sc(x_p, si, di)
    return out.reshape(-1, 1)


# speedup vs baseline: 1.0936x; 1.0003x over previous
import jax
import jax.numpy as jnp
from jax import lax
from jax.experimental import pallas as pl
from jax.experimental.pallas import tpu as pltpu
from jax.experimental.pallas import tpu_sc as plsc

N_NODES = 10000
D = 256
N_EDGES = 160000
NC = 2
NS = 16
L = 16
NW = NC * NS
DP = 128
GSZ = 32
NBUF = 4
EPW = 5120
E_PAD = EPW * NW
G = EPW // GSZ
NOUT = G // NBUF
EPW_LAST = N_EDGES - (NW - 1) * EPW


N_SP = 10240
ROWS_PER_TILE = N_SP // NS


def _sc_body(x_hbm, si_hbm, di_hbm, out_hbm,
             si_v, di_v, x_sp, sbuf, dbuf, cs_v, out_v, sems, stage_sem):
    sid = lax.axis_index("s")
    wid = sid * NC + lax.axis_index("c")
    base = wid * EPW
    stripe = pltpu.async_copy(
        x_hbm.at[pl.ds(sid * ROWS_PER_TILE, ROWS_PER_TILE)],
        x_sp.at[pl.ds(sid * ROWS_PER_TILE, ROWS_PER_TILE)], stage_sem)
    pltpu.sync_copy(si_hbm.at[pl.ds(base, EPW)], si_v)
    pltpu.sync_copy(di_hbm.at[pl.ds(base, EPW)], di_v)

    def fire(g, b, src=None):
        src = x_sp if src is None else src
        pltpu.async_copy(src.at[si_v.at[pl.ds(g * GSZ, GSZ)]],
                         sbuf.at[b], sems.at[b, 0])
        pltpu.async_copy(src.at[di_v.at[pl.ds(g * GSZ, GSZ)]],
                         dbuf.at[b], sems.at[b, 1])

    def wait(g, b):
        pltpu.make_async_copy(x_sp.at[si_v.at[pl.ds(g * GSZ, GSZ)]],
                              sbuf.at[b], sems.at[b, 0]).wait()
        pltpu.make_async_copy(x_sp.at[di_v.at[pl.ds(g * GSZ, GSZ)]],
                              dbuf.at[b], sems.at[b, 1]).wait()

    for b in range(NBUF):
        fire(b, b, src=x_hbm)
    stripe.wait()
    plsc.subcore_barrier()

    row_sel = lax.iota(jnp.int32, L)

    def outer(i, carry):
        for b in range(NBUF):
            g = i * NBUF + b
            wait(g, b)
            for sg in range(GSZ // L):
                for ee in range(L):
                    e = sg * L + ee
                    acc0 = acc1 = None
                    for c in range(DP // L):
                        s = plsc.bitcast(sbuf[b, e, pl.ds(c * L, L)],
                                         jnp.bfloat16)
                        d = plsc.bitcast(dbuf[b, e, pl.ds(c * L, L)],
                                         jnp.bfloat16)
                        p0, p1 = plsc.unpack(
                            s * d, format=plsc.PackFormat.INTERLEAVED)
                        acc0 = p0 if acc0 is None else acc0 + p0
                        acc1 = p1 if acc1 is None else acc1 + p1
                    cs_v[pl.ds(ee * L, L)] = acc0 + acc1
                cols = [plsc.load_gather(cs_v, [row_sel * L + c])
                        for c in range(L)]
                while len(cols) > 1:
                    cols = [a + bb for a, bb in zip(cols[::2], cols[1::2])]
                out_v[pl.ds(g * GSZ + sg * L, L)] = cols[0]
            ng = g + NBUF

            @pl.when(ng < G)
            def _():
                fire(ng, b)
        return carry

    lax.fori_loop(0, NOUT, outer, 0)

    @pl.when(wid < NW - 1)
    def _():
        pltpu.sync_copy(out_v, out_hbm.at[pl.ds(base, EPW)])

    @pl.when(wid == NW - 1)
    def _():
        pltpu.sync_copy(out_v.at[pl.ds(0, EPW_LAST)],
                        out_hbm.at[pl.ds(base, EPW_LAST)])


def _run_sc(x, si, di):
    return pl.kernel(
        _sc_body,
        out_type=jax.ShapeDtypeStruct((N_EDGES,), jnp.float32),
        mesh=plsc.VectorSubcoreMesh(core_axis_name="c", subcore_axis_name="s",
                                    num_cores=NC, num_subcores=NS),
        compiler_params=pltpu.CompilerParams(needs_layout_passes=False),
        scratch_types=[
            pltpu.VMEM((EPW,), jnp.int32),
            pltpu.VMEM((EPW,), jnp.int32),
            pltpu.VMEM_SHARED((N_SP, DP), jnp.int32),
            pltpu.VMEM((NBUF, GSZ, DP), jnp.int32),
            pltpu.VMEM((NBUF, GSZ, DP), jnp.int32),
            pltpu.VMEM((L * L,), jnp.float32),
            pltpu.VMEM((EPW,), jnp.float32),
            pltpu.SemaphoreType.DMA((NBUF, 2)),
            pltpu.SemaphoreType.DMA,
        ],
    )(x, si, di)


@jax.jit
def kernel(x, edge_label_index):
    u = lax.bitcast_convert_type(x, jnp.uint32)

    def _rne_hi16(v):
        return (v + 0x7FFF + ((v >> 16) & 1)) >> 16

    lo = _rne_hi16(u[:, :DP])
    hi = _rne_hi16(u[:, DP:])
    x_p = lax.bitcast_convert_type(lo | (hi << 16), jnp.int32)
    x_p = jnp.pad(x_p, ((0, N_SP - N_NODES), (0, 0)))
    eli = edge_label_index.astype(jnp.int32)
    pad = E_PAD - N_EDGES
    si = jnp.pad(eli[0], (0, pad))
    di = jnp.pad(eli[1], (0, pad))
    out = _run_sc(x_p, si, di)
    return out.reshape(-1, 1)
